# Initial kernel scaffold; baseline (speedup 1.0000x reference)
import jax
import jax.numpy as jnp
from jax.experimental import pallas as pl


def kernel(term, table):
    raise NotImplementedError("write your pallas kernel here")


# speedup vs baseline: 1.4550x; 1.4550x over previous
import functools

import jax
import jax.numpy as jnp
from jax import lax
from jax.experimental import pallas as pl
from jax.experimental.pallas import tpu as pltpu
from jax.experimental.pallas import tpu_sc as plsc

CHUNK = 1024


def _gather_sc(term_flat, table):
    B = term_flat.shape[0]
    D = table.shape[1]
    info = plsc.get_sparse_core_info()
    NC, NS = info.num_cores, info.num_subcores
    NW = NC * NS
    b_per_w = B // NW
    n_chunks = b_per_w // CHUNK
    assert b_per_w % CHUNK == 0 and B % NW == 0

    mesh = plsc.VectorSubcoreMesh(core_axis_name="c", subcore_axis_name="s")

    @functools.partial(
        pl.kernel,
        mesh=mesh,
        out_type=jax.ShapeDtypeStruct((B, D), jnp.float32),
        scratch_types=[
            pltpu.VMEM((CHUNK,), jnp.int32),
            pltpu.VMEM((CHUNK, D), jnp.float32),
            pltpu.SemaphoreType.DMA,
        ],
        compiler_params=pltpu.CompilerParams(use_tc_tiling_on_sc=False),
    )
    def k(term_hbm, table_hbm, out_hbm, idx_v, rows_v, sem):
        wid = lax.axis_index("s") * NC + lax.axis_index("c")
        base = wid * b_per_w

        def body(c, _):
            off = base + c * CHUNK
            pltpu.sync_copy(term_hbm.at[pl.ds(off, CHUNK)], idx_v)
            pltpu.async_copy(table_hbm.at[idx_v], rows_v, sem).wait()
            pltpu.sync_copy(rows_v, out_hbm.at[pl.ds(off, CHUNK)])
            return ()

        lax.fori_loop(0, n_chunks, body, ())

    return k(term_flat, table)


def _mask_tc(term):
    def mk(t_ref, o_ref):
        o_ref[...] = t_ref[...] == 0

    return pl.pallas_call(
        mk,
        out_shape=jax.ShapeDtypeStruct(term.shape, jnp.bool_),
    )(term)


@jax.jit
def kernel(term, table):
    bsz, hist = term.shape
    term_flat = term.reshape(bsz * hist)
    emb = _gather_sc(term_flat, table)
    mask = _mask_tc(term)
    return emb.reshape(bsz, hist, table.shape[1]), mask


# speedup vs baseline: 1.4961x; 1.0282x over previous
import functools

import jax
import jax.numpy as jnp
from jax import lax
from jax.experimental import pallas as pl
from jax.experimental.pallas import tpu as pltpu
from jax.experimental.pallas import tpu_sc as plsc

CHUNK = 1024
NSLOT = 3


def _gather_sc(term_flat, table):
    B = term_flat.shape[0]
    D = table.shape[1]
    info = plsc.get_sparse_core_info()
    NC, NS = info.num_cores, info.num_subcores
    NW = NC * NS
    b_per_w = B // NW
    n_chunks = b_per_w // CHUNK
    assert b_per_w % CHUNK == 0 and B % NW == 0

    mesh = plsc.VectorSubcoreMesh(core_axis_name="c", subcore_axis_name="s")

    scratch = (
        [pltpu.VMEM((CHUNK,), jnp.int32) for _ in range(NSLOT)]
        + [pltpu.VMEM((CHUNK, D), jnp.float32) for _ in range(NSLOT)]
        + [pltpu.SemaphoreType.DMA for _ in range(3 * NSLOT)]
    )

    @functools.partial(
        pl.kernel,
        mesh=mesh,
        out_type=jax.ShapeDtypeStruct((B, D), jnp.float32),
        scratch_types=scratch,
        compiler_params=pltpu.CompilerParams(use_tc_tiling_on_sc=False),
    )
    def k(term_hbm, table_hbm, out_hbm, *refs):
        idx_v = refs[0:NSLOT]
        rows_v = refs[NSLOT : 2 * NSLOT]
        sem_i = refs[2 * NSLOT : 2 * NSLOT + NSLOT]
        sem_g = refs[2 * NSLOT + NSLOT : 2 * NSLOT + 2 * NSLOT]
        sem_w = refs[2 * NSLOT + 2 * NSLOT : 2 * NSLOT + 3 * NSLOT]

        wid = lax.axis_index("s") * NC + lax.axis_index("c")
        base = wid * b_per_w

        def copy_i(c):
            s = c % NSLOT
            return pltpu.make_async_copy(
                term_hbm.at[pl.ds(base + c * CHUNK, CHUNK)], idx_v[s], sem_i[s]
            )

        def copy_g(c):
            s = c % NSLOT
            return pltpu.make_async_copy(
                table_hbm.at[idx_v[s]], rows_v[s], sem_g[s]
            )

        def copy_w(c):
            s = c % NSLOT
            return pltpu.make_async_copy(
                rows_v[s], out_hbm.at[pl.ds(base + c * CHUNK, CHUNK)], sem_w[s]
            )

        for c in range(min(NSLOT, n_chunks)):
            copy_i(c).start()
        for c in range(n_chunks):
            s = c % NSLOT
            copy_i(c).wait()
            if c >= NSLOT:
                copy_w(c - NSLOT).wait()
            copy_g(c).start()
            if c >= 1:
                copy_g(c - 1).wait()
                copy_w(c - 1).start()
                if c + 2 < n_chunks:
                    copy_i(c + 2).start()
        copy_g(n_chunks - 1).wait()
        copy_w(n_chunks - 1).start()
        for c in range(max(0, n_chunks - NSLOT), n_chunks):
            copy_w(c).wait()

    return k(term_flat, table)


def _mask_tc(term):
    def mk(t_ref, o_ref):
        o_ref[...] = t_ref[...] == 0

    return pl.pallas_call(
        mk,
        out_shape=jax.ShapeDtypeStruct(term.shape, jnp.bool_),
    )(term)


@jax.jit
def kernel(term, table):
    bsz, hist = term.shape
    term_flat = term.reshape(bsz * hist)
    emb = _gather_sc(term_flat, table)
    mask = _mask_tc(term)
    return emb.reshape(bsz, hist, table.shape[1]), mask
